# baseline (speedup 1.0000x reference)
import functools

import jax
import jax.numpy as jnp
from jax.experimental import pallas as pl
from jax.experimental.pallas import tpu as pltpu

_K = 5
_TOPN = _K + 1
_DEPTH = 8
_NEG = float("-inf")


def _extract_top(cat, n):
    cat_cols = jax.lax.broadcasted_iota(jnp.int32, cat.shape, 1)
    big = jnp.int32(2**31 - 1)
    outs = []
    for _ in range(n):
        v = jnp.max(cat, axis=1, keepdims=True)
        hit = cat == v
        idx = jnp.min(jnp.where(hit, cat_cols, big), axis=1, keepdims=True)
        cat = jnp.where(cat_cols == idx, _NEG, cat)
        outs.append(v)
    return jnp.concatenate(outs, axis=1)


def _insert(tup, v):
    for kk in range(len(tup)):
        hi = jnp.maximum(tup[kk], v)
        v = jnp.minimum(tup[kk], v)
        tup[kk] = hi
    return tup


def _merge_halves(tup):
    d = len(tup)
    h = tup[0].shape[0] // 2
    a = [t[:h] for t in tup]
    b = [t[h:] for t in tup]
    m = [jnp.maximum(a[i], b[d - 1 - i]) for i in range(d)]
    for dist in (4, 2, 1):
        for i in range(d):
            if (i // dist) % 2 == 0:
                hi = jnp.maximum(m[i], m[i + dist])
                lo = jnp.minimum(m[i], m[i + dist])
                m[i], m[i + dist] = hi, lo
    return m


def _body(yt_ref, ys_ref, yl_ref, x_ref, out_ref, *, nt, w, total_rows):
    i = pl.program_id(0)

    nu = 10
    while nt % nu:
        nu -= 1

    def step(j, carry):
        tup = list(carry)
        for c in range(nu):
            tup = _insert(tup, x_ref[0, nu * j + c, :, :])
        return tuple(tup)

    init = tuple(jnp.full((8, w), _NEG, jnp.float32) for _ in range(_DEPTH))
    res = list(jax.lax.fori_loop(0, nt // nu, step, init))

    res = _merge_halves(res)
    res = _merge_halves(res)
    res = _merge_halves(res)
    cand = jnp.concatenate(res, axis=1)
    top = _extract_top(cand, _DEPTH - 1)

    tsl = x_ref[0, yt_ref[0, 0, 0], :, :]
    subi = jax.lax.broadcasted_iota(jnp.int32, (8, w), 0)
    lani = jax.lax.broadcasted_iota(jnp.int32, (8, w), 1)
    ist = (subi == ys_ref[0, 0, 0]) & (lani == yl_ref[0, 0, 0])
    t = jnp.sum(jnp.where(ist, tsl, 0.0)).reshape(1, 1)

    m = top[:, :1]
    xb = x_ref[...]
    e = jnp.exp(xb - m.reshape(1, 1, 1, 1))
    s = jnp.sum(jnp.sum(jnp.sum(e, axis=3), axis=2), axis=1).reshape(1, 1)

    l6 = top[:, _TOPN:_TOPN + 1]
    dropval = jnp.where(t >= l6, t, l6)
    cols7 = jax.lax.broadcasted_iota(jnp.int32, (1, _DEPTH - 1), 1)
    hit = top == dropval
    dropidx = jnp.min(jnp.where(hit, cols7, jnp.int32(2**31 - 1)),
                      axis=1, keepdims=True)
    keep = cols7 != dropidx

    pp = jnp.exp(t - m) / s
    pn = jnp.exp(top - m) / s
    terms = (1.0 + pn - pp) ** 2
    loss = jnp.sum(jnp.where(keep, terms, 0.0), axis=1, keepdims=True) / _K
    part = loss / total_rows

    @pl.when(i == 0)
    def _init_out():
        out_ref[...] = jnp.zeros((1, 1), jnp.float32)

    out_ref[...] += part


@functools.partial(jax.jit, static_argnames=("w",))
def _run(pred, y2, w):
    rows, nclass = pred.shape
    tile = 8 * w
    nt = nclass // tile
    pred4 = pred.reshape(rows, nt, 8, w)
    yt = (y2 // tile).reshape(rows, 1, 1)
    ys = ((y2 % tile) // w).reshape(rows, 1, 1)
    yl = (y2 % w).reshape(rows, 1, 1)
    body = functools.partial(_body, nt=nt, w=w, total_rows=rows)
    smem = functools.partial(pl.BlockSpec, memory_space=pltpu.SMEM)
    out = pl.pallas_call(
        body,
        grid=(rows,),
        in_specs=[
            smem((1, 1, 1), lambda i: (i, 0, 0)),
            smem((1, 1, 1), lambda i: (i, 0, 0)),
            smem((1, 1, 1), lambda i: (i, 0, 0)),
            pl.BlockSpec((1, nt, 8, w), lambda i: (i, 0, 0, 0)),
        ],
        out_specs=pl.BlockSpec((1, 1), lambda i: (0, 0)),
        out_shape=jax.ShapeDtypeStruct((1, 1), jnp.float32),
        compiler_params=pltpu.CompilerParams(
            dimension_semantics=("arbitrary",)),
    )(yt, ys, yl, pred4)
    return out[0, 0]


def kernel(pred, y, epoch=0):
    rows, nclass = pred.shape
    for cand in (250, 125, 100, 50, 25, 20, 10, 5, 4, 2, 1):
        if nclass % (cand * 8) == 0:
            w = cand
            break
    else:
        w = nclass
    y2 = y.reshape(rows, 1).astype(jnp.int32)
    return _run(pred, y2, w)


# speedup vs baseline: 1.7901x; 1.7901x over previous
import functools

import jax
import jax.numpy as jnp
from jax.experimental import pallas as pl
from jax.experimental.pallas import tpu as pltpu

_K = 5
_TOPN = _K + 1
_DEPTH = _K + 2
_NEG = float("-inf")


def _extract_top(cat, n):
    cat_cols = jax.lax.broadcasted_iota(jnp.int32, cat.shape, 1)
    big = jnp.int32(2**31 - 1)
    outs = []
    for _ in range(n):
        v = jnp.max(cat, axis=1, keepdims=True)
        hit = cat == v
        idx = jnp.min(jnp.where(hit, cat_cols, big), axis=1, keepdims=True)
        cat = jnp.where(cat_cols == idx, _NEG, cat)
        outs.append(v)
    return jnp.concatenate(outs, axis=1)


_SORT8_NET = (
    (0, 1), (2, 3), (4, 5), (6, 7),
    (0, 2), (1, 3), (4, 6), (5, 7),
    (1, 2), (5, 6),
    (0, 4), (1, 5), (2, 6), (3, 7),
    (2, 4), (3, 5),
    (1, 2), (3, 4), (5, 6),
)


def _sort8(vs):
    vs = list(vs)
    for a, b in _SORT8_NET:
        hi = jnp.maximum(vs[a], vs[b])
        lo = jnp.minimum(vs[a], vs[b])
        vs[a], vs[b] = hi, lo
    return vs


def _merge8(tup, s):
    m = [jnp.maximum(s[i], tup[7 - i]) for i in range(8)]
    for dist in (4, 2, 1):
        for i in range(8):
            if (i // dist) % 2 == 0:
                hi = jnp.maximum(m[i], m[i + dist])
                lo = jnp.minimum(m[i], m[i + dist])
                m[i], m[i + dist] = hi, lo
    return m


def _body(yhi_ref, ylo_ref, x_ref, out_ref, *, rows, nf, w, total_rows):
    i = pl.program_id(0)
    ylo = ylo_ref[...]

    unroll = 16
    def step(j, carry):
        tup = list(carry)
        for c in range(0, unroll, 8):
            batch = [x_ref[:, unroll * j + c + q, :] for q in range(8)]
            tup = _merge8(tup, _sort8(batch))
        return tuple(tup)

    init = tuple(jnp.full((rows, w), _NEG, jnp.float32) for _ in range(8))
    res = jax.lax.fori_loop(0, nf // unroll, step, init)

    cand = jnp.concatenate(res, axis=1)
    top7 = _extract_top(cand, _DEPTH)

    lane = jax.lax.broadcasted_iota(jnp.int32, (rows, w), 1)
    tmat = jnp.concatenate(
        [x_ref[r, yhi_ref[r, 0], :].reshape(1, w) for r in range(rows)],
        axis=0)
    t = jnp.sum(jnp.where(lane == ylo, tmat, 0.0), axis=1, keepdims=True)

    m = top7[:, :1]
    m3 = m.reshape(rows, 1, 1)
    xb = x_ref[...]
    s = jnp.sum(jnp.sum(jnp.exp(xb - m3), axis=2), axis=1).reshape(rows, 1)

    l6 = top7[:, _TOPN:]
    dropval = jnp.where(t >= l6, t, l6)
    cols7 = jax.lax.broadcasted_iota(jnp.int32, (rows, _DEPTH), 1)
    hit = top7 == dropval
    dropidx = jnp.min(jnp.where(hit, cols7, jnp.int32(2**31 - 1)),
                      axis=1, keepdims=True)
    keep = cols7 != dropidx

    pp = jnp.exp(t - m) / s
    pn = jnp.exp(top7 - m) / s
    terms = (1.0 + pn - pp) ** 2
    loss = jnp.sum(jnp.where(keep, terms, 0.0), axis=1, keepdims=True) / _K
    part = (jnp.sum(loss) / total_rows).reshape(1, 1)

    @pl.when(i == 0)
    def _init_out():
        out_ref[...] = jnp.zeros((1, 1), jnp.float32)

    out_ref[...] += part


@functools.partial(jax.jit, static_argnames=("w", "rblk"))
def _run(pred, y2, w, rblk):
    rows, nclass = pred.shape
    nf = nclass // w
    pred3 = pred.reshape(rows, nf, w)
    yhi = y2 // w
    ylo = y2 % w
    body = functools.partial(_body, rows=rblk, nf=nf, w=w, total_rows=rows)
    out = pl.pallas_call(
        body,
        grid=(rows // rblk,),
        in_specs=[
            pl.BlockSpec((rblk, 1), lambda i: (i, 0),
                         memory_space=pltpu.SMEM),
            pl.BlockSpec((rblk, 1), lambda i: (i, 0)),
            pl.BlockSpec((rblk, nf, w), lambda i: (i, 0, 0)),
        ],
        out_specs=pl.BlockSpec((1, 1), lambda i: (0, 0)),
        out_shape=jax.ShapeDtypeStruct((1, 1), jnp.float32),
        compiler_params=pltpu.CompilerParams(
            dimension_semantics=("arbitrary",)),
    )(yhi, ylo, pred3)
    return out[0, 0]


def kernel(pred, y, epoch=0):
    rows, nclass = pred.shape
    for cand in (250, 125, 100, 50, 25, 20, 10, 5, 4, 2, 1):
        if nclass % (cand * 16) == 0:
            w = cand
            break
    else:
        w = nclass
    rblk = 8 if rows % 8 == 0 else rows
    y2 = y.reshape(rows, 1).astype(jnp.int32)
    return _run(pred, y2, w, rblk)


# speedup vs baseline: 2.6388x; 1.4741x over previous
import functools

import jax
import jax.numpy as jnp
from jax.experimental import pallas as pl
from jax.experimental.pallas import tpu as pltpu

_K = 5
_TOPN = _K + 1
_DEPTH = _K + 2
_NEG = float("-inf")


def _extract_top(cat, n):
    cat_cols = jax.lax.broadcasted_iota(jnp.int32, cat.shape, 1)
    big = jnp.int32(2**31 - 1)
    outs = []
    for _ in range(n):
        v = jnp.max(cat, axis=1, keepdims=True)
        hit = cat == v
        idx = jnp.min(jnp.where(hit, cat_cols, big), axis=1, keepdims=True)
        cat = jnp.where(cat_cols == idx, _NEG, cat)
        outs.append(v)
    return jnp.concatenate(outs, axis=1)


_SORT8_NET = (
    (0, 1), (2, 3), (4, 5), (6, 7),
    (0, 2), (1, 3), (4, 6), (5, 7),
    (1, 2), (5, 6),
    (0, 4), (1, 5), (2, 6), (3, 7),
    (2, 4), (3, 5),
    (1, 2), (3, 4), (5, 6),
)


def _sort8(vs):
    vs = list(vs)
    for a, b in _SORT8_NET:
        hi = jnp.maximum(vs[a], vs[b])
        lo = jnp.minimum(vs[a], vs[b])
        vs[a], vs[b] = hi, lo
    return vs


def _merge8(tup, s):
    m = [jnp.maximum(s[i], tup[7 - i]) for i in range(8)]
    for dist in (4, 2, 1):
        for i in range(8):
            if (i // dist) % 2 == 0:
                hi = jnp.maximum(m[i], m[i + dist])
                lo = jnp.minimum(m[i], m[i + dist])
                m[i], m[i + dist] = hi, lo
    return m


def _body(yhi_ref, ylo_ref, x_ref, out_ref, *, rows, nf, w, total_rows):
    i = pl.program_id(0)
    ylo = ylo_ref[...]

    unroll = 16
    def step(j, carry):
        tup = list(carry)
        for c in range(0, unroll, 8):
            batch = [x_ref[unroll * j + c + q, :, :] for q in range(8)]
            tup = _merge8(tup, _sort8(batch))
        return tuple(tup)

    init = tuple(jnp.full((rows, w), _NEG, jnp.float32) for _ in range(8))
    res = jax.lax.fori_loop(0, nf // unroll, step, init)

    cand = jnp.concatenate(res, axis=1)
    top7 = _extract_top(cand, _DEPTH)

    lane = jax.lax.broadcasted_iota(jnp.int32, (rows, w), 1)
    tmat = jnp.concatenate(
        [x_ref[yhi_ref[r, 0], r, :].reshape(1, w) for r in range(rows)],
        axis=0)
    t = jnp.sum(jnp.where(lane == ylo, tmat, 0.0), axis=1, keepdims=True)

    m = top7[:, :1]
    m3 = m.reshape(1, rows, 1)
    xb = x_ref[...]
    s = jnp.sum(jnp.sum(jnp.exp(xb - m3), axis=2), axis=0).reshape(rows, 1)

    l6 = top7[:, _TOPN:]
    dropval = jnp.where(t >= l6, t, l6)
    cols7 = jax.lax.broadcasted_iota(jnp.int32, (rows, _DEPTH), 1)
    hit = top7 == dropval
    dropidx = jnp.min(jnp.where(hit, cols7, jnp.int32(2**31 - 1)),
                      axis=1, keepdims=True)
    keep = cols7 != dropidx

    pp = jnp.exp(t - m) / s
    pn = jnp.exp(top7 - m) / s
    terms = (1.0 + pn - pp) ** 2
    loss = jnp.sum(jnp.where(keep, terms, 0.0), axis=1, keepdims=True) / _K
    part = (jnp.sum(loss) / total_rows).reshape(1, 1)

    @pl.when(i == 0)
    def _init_out():
        out_ref[...] = jnp.zeros((1, 1), jnp.float32)

    out_ref[...] += part


@functools.partial(jax.jit, static_argnames=("w", "rblk"))
def _run(pred, y2, w, rblk):
    rows, nclass = pred.shape
    nf = nclass // w
    pred3 = jnp.transpose(pred.reshape(rows, nf, w), (1, 0, 2))
    yhi = y2 // w
    ylo = y2 % w
    body = functools.partial(_body, rows=rblk, nf=nf, w=w, total_rows=rows)
    out = pl.pallas_call(
        body,
        grid=(rows // rblk,),
        in_specs=[
            pl.BlockSpec((rblk, 1), lambda i: (i, 0),
                         memory_space=pltpu.SMEM),
            pl.BlockSpec((rblk, 1), lambda i: (i, 0)),
            pl.BlockSpec((nf, rblk, w), lambda i: (0, i, 0)),
        ],
        out_specs=pl.BlockSpec((1, 1), lambda i: (0, 0)),
        out_shape=jax.ShapeDtypeStruct((1, 1), jnp.float32),
        compiler_params=pltpu.CompilerParams(
            dimension_semantics=("arbitrary",)),
    )(yhi, ylo, pred3)
    return out[0, 0]


def kernel(pred, y, epoch=0):
    rows, nclass = pred.shape
    for cand in (250, 125, 100, 50, 25, 20, 10, 5, 4, 2, 1):
        if nclass % (cand * 16) == 0:
            w = cand
            break
    else:
        w = nclass
    rblk = 8 if rows % 8 == 0 else rows
    y2 = y.reshape(rows, 1).astype(jnp.int32)
    return _run(pred, y2, w, rblk)


# speedup vs baseline: 2.6877x; 1.0185x over previous
import functools

import jax
import jax.numpy as jnp
from jax.experimental import pallas as pl
from jax.experimental.pallas import tpu as pltpu

_K = 5
_TOPN = _K + 1
_DEPTH = _K + 2
_NEG = float("-inf")


def _extract_top(cat, n):
    cat_cols = jax.lax.broadcasted_iota(jnp.int32, cat.shape, 1)
    big = jnp.int32(2**31 - 1)
    outs = []
    for _ in range(n):
        v = jnp.max(cat, axis=1, keepdims=True)
        hit = cat == v
        idx = jnp.min(jnp.where(hit, cat_cols, big), axis=1, keepdims=True)
        cat = jnp.where(cat_cols == idx, _NEG, cat)
        outs.append(v)
    return jnp.concatenate(outs, axis=1)


_SORT8_NET = (
    (0, 1), (2, 3), (4, 5), (6, 7),
    (0, 2), (1, 3), (4, 6), (5, 7),
    (1, 2), (5, 6),
    (0, 4), (1, 5), (2, 6), (3, 7),
    (2, 4), (3, 5),
    (1, 2), (3, 4), (5, 6),
)


def _sort8(vs):
    vs = list(vs)
    for a, b in _SORT8_NET:
        hi = jnp.maximum(vs[a], vs[b])
        lo = jnp.minimum(vs[a], vs[b])
        vs[a], vs[b] = hi, lo
    return vs


def _merge8(tup, s):
    m = [jnp.maximum(s[i], tup[7 - i]) for i in range(8)]
    for dist in (4, 2, 1):
        for i in range(8):
            if (i // dist) % 2 == 0:
                hi = jnp.maximum(m[i], m[i + dist])
                lo = jnp.minimum(m[i], m[i + dist])
                m[i], m[i + dist] = hi, lo
    return m


def _body(yhi_ref, ylo_ref, x_ref, out_ref, *, rows, nf, w, total_rows):
    i = pl.program_id(0)
    ylo = ylo_ref[...]

    unroll = 40
    def step(j, carry):
        tup = list(carry)
        for c in range(0, unroll, 8):
            batch = [x_ref[unroll * j + c + q, :, :] for q in range(8)]
            tup = _merge8(tup, _sort8(batch))
        return tuple(tup)

    init = tuple(jnp.full((rows, w), _NEG, jnp.float32) for _ in range(8))
    res = jax.lax.fori_loop(0, nf // unroll, step, init)

    cand = jnp.concatenate(res, axis=1)
    top7 = _extract_top(cand, _DEPTH)

    lane = jax.lax.broadcasted_iota(jnp.int32, (rows, w), 1)
    tmat = jnp.concatenate(
        [x_ref[yhi_ref[r, 0], r, :].reshape(1, w) for r in range(rows)],
        axis=0)
    t = jnp.sum(jnp.where(lane == ylo, tmat, 0.0), axis=1, keepdims=True)

    m = top7[:, :1]
    m3 = m.reshape(1, rows, 1)
    xb = x_ref[...]
    s = jnp.sum(jnp.sum(jnp.exp(xb - m3), axis=2), axis=0).reshape(rows, 1)

    l6 = top7[:, _TOPN:]
    dropval = jnp.where(t >= l6, t, l6)
    cols7 = jax.lax.broadcasted_iota(jnp.int32, (rows, _DEPTH), 1)
    hit = top7 == dropval
    dropidx = jnp.min(jnp.where(hit, cols7, jnp.int32(2**31 - 1)),
                      axis=1, keepdims=True)
    keep = cols7 != dropidx

    pp = jnp.exp(t - m) / s
    pn = jnp.exp(top7 - m) / s
    terms = (1.0 + pn - pp) ** 2
    loss = jnp.sum(jnp.where(keep, terms, 0.0), axis=1, keepdims=True) / _K
    part = (jnp.sum(loss) / total_rows).reshape(1, 1)

    @pl.when(i == 0)
    def _init_out():
        out_ref[...] = jnp.zeros((1, 1), jnp.float32)

    out_ref[...] += part


@functools.partial(jax.jit, static_argnames=("w", "rblk"))
def _run(pred, y2, w, rblk):
    rows, nclass = pred.shape
    nf = nclass // w
    pred3 = jnp.transpose(pred.reshape(rows, nf, w), (1, 0, 2))
    yhi = y2 // w
    ylo = y2 % w
    body = functools.partial(_body, rows=rblk, nf=nf, w=w, total_rows=rows)
    out = pl.pallas_call(
        body,
        grid=(rows // rblk,),
        in_specs=[
            pl.BlockSpec((rblk, 1), lambda i: (i, 0),
                         memory_space=pltpu.SMEM),
            pl.BlockSpec((rblk, 1), lambda i: (i, 0)),
            pl.BlockSpec((nf, rblk, w), lambda i: (0, i, 0)),
        ],
        out_specs=pl.BlockSpec((1, 1), lambda i: (0, 0)),
        out_shape=jax.ShapeDtypeStruct((1, 1), jnp.float32),
        compiler_params=pltpu.CompilerParams(
            dimension_semantics=("arbitrary",)),
    )(yhi, ylo, pred3)
    return out[0, 0]


def kernel(pred, y, epoch=0):
    rows, nclass = pred.shape
    for cand in (250, 125, 100, 50, 25, 20, 10, 5, 4, 2, 1):
        if nclass % (cand * 40) == 0:
            w = cand
            break
    else:
        w = nclass
    rblk = 8 if rows % 8 == 0 else rows
    y2 = y.reshape(rows, 1).astype(jnp.int32)
    return _run(pred, y2, w, rblk)
